# baseline (speedup 1.0000x reference)
import functools

import jax
import jax.numpy as jnp
from jax import lax
from jax.experimental import pallas as pl
from jax.experimental.pallas import tpu as pltpu
from jax.experimental.pallas import tpu_sc as plsc

D = 128
W = D // 2
NV = 10000
NE = 320000

_info = plsc.get_sparse_core_info()
NC, NS, L = _info.num_cores, _info.num_subcores, _info.num_lanes
NW = NC * NS
EPW = NE // NW
C = 80
C2 = 2 * C
NCHUNK = EPW // C
G = C // L

_mesh = plsc.VectorSubcoreMesh(core_axis_name="c", subcore_axis_name="s")


@functools.partial(
    pl.kernel,
    mesh=_mesh,
    compiler_params=pltpu.CompilerParams(
        use_tc_tiling_on_sc=False, needs_layout_passes=False
    ),
    out_type=jax.ShapeDtypeStruct((NE,), jnp.float32),
    scratch_types=[
        pltpu.VMEM((EPW,), jnp.int32),
        pltpu.VMEM((EPW,), jnp.int32),
        pltpu.VMEM((2 * EPW,), jnp.int32),
        pltpu.VMEM((C2, W), jnp.int32),
        pltpu.VMEM((C2, W), jnp.int32),
        pltpu.VMEM((EPW,), jnp.float32),
        pltpu.SemaphoreType.DMA,
        pltpu.SemaphoreType.DMA,
    ],
)
def _score_kernel(table_hbm, ei_hbm, out_hbm, sidx, didx, cidx, ra, rb,
                  sc, sem_a, sem_b):
    wid = lax.axis_index("s") * NC + lax.axis_index("c")
    ebase = wid * EPW
    lane = lax.iota(jnp.int32, L)

    pltpu.sync_copy(ei_hbm.at[pl.ds(ebase, EPW)], sidx)
    pltpu.sync_copy(ei_hbm.at[pl.ds(NE + ebase, EPW)], didx)

    def mix(i, c):
        ci = i // G
        j = i % G
        src_off = ci * C + j * L
        dst_off = ci * C2 + j * L
        cidx[pl.ds(dst_off, L)] = sidx[pl.ds(src_off, L)]
        cidx[pl.ds(dst_off + C, L)] = didx[pl.ds(src_off, L)]
        return c

    lax.fori_loop(0, NCHUNK * G, mix, 0, unroll=4)

    def issue(ci, rows, sem):
        pltpu.async_copy(table_hbm.at[cidx.at[pl.ds(ci * C2, C2)]], rows, sem)

    def wait(ci, rows, sem):
        pltpu.make_async_copy(
            table_hbm.at[cidx.at[pl.ds(ci * C2, C2)]], rows, sem).wait()

    def compute(ci, rows_ref):
        def grp(g, c):
            srows = g * L + lane
            trows = srows + C

            def prod(w):
                col = w ^ lane
                s32 = plsc.load_gather(rows_ref, [srows, col])
                t32 = plsc.load_gather(rows_ref, [trows, col])
                sbf = plsc.bitcast(s32, jnp.bfloat16)
                tbf = plsc.bitcast(t32, jnp.bfloat16)
                p_a, p_b = plsc.unpack(sbf * tbf,
                                       format=plsc.PackFormat.INTERLEAVED)
                return p_a + p_b

            def wstep(w4, accs):
                a0, a1, a2, a3 = accs
                w = w4 * 4
                return (a0 + prod(w), a1 + prod(w + 1),
                        a2 + prod(w + 2), a3 + prod(w + 3))

            z = jnp.zeros((L,), jnp.float32)
            a0, a1, a2, a3 = lax.fori_loop(0, W // 4, wstep, (z, z, z, z),
                                           unroll=4)
            sc[pl.ds(ci * C + g * L, L)] = (a0 + a1) + (a2 + a3)
            return c

        lax.fori_loop(0, G, grp, 0)

    issue(0, ra, sem_a)

    def pair(k, carry):
        c0 = 2 * k
        issue(c0 + 1, rb, sem_b)
        wait(c0, ra, sem_a)
        compute(c0, ra)
        issue(c0 + 2, ra, sem_a)
        wait(c0 + 1, rb, sem_b)
        compute(c0 + 1, rb)
        return carry

    lax.fori_loop(0, (NCHUNK - 1) // 2, pair, 0)
    wait(NCHUNK - 1, ra, sem_a)
    compute(NCHUNK - 1, ra)

    pltpu.sync_copy(sc, out_hbm.at[pl.ds(ebase, EPW)])


def kernel(gnn_emb, edge_index):
    mu16 = gnn_emb[:, :D].astype(jnp.bfloat16)
    table = lax.bitcast_convert_type(mu16.reshape(NV, W, 2), jnp.int32)
    ei = edge_index.reshape(2 * NE)
    return _score_kernel(table, ei)


# speedup vs baseline: 1.1160x; 1.1160x over previous
import functools

import jax
import jax.numpy as jnp
from jax import lax
from jax.experimental import pallas as pl
from jax.experimental.pallas import tpu as pltpu
from jax.experimental.pallas import tpu_sc as plsc

D = 128
W = D // 2
NV = 10000
NE = 320000

_info = plsc.get_sparse_core_info()
NC, NS, L = _info.num_cores, _info.num_subcores, _info.num_lanes
NW = NC * NS
EPW = NE // NW
C = 80
H = C // 2
NCHUNK = EPW // C
G = C // L

_mesh = plsc.VectorSubcoreMesh(core_axis_name="c", subcore_axis_name="s")


@functools.partial(
    pl.kernel,
    mesh=_mesh,
    compiler_params=pltpu.CompilerParams(
        use_tc_tiling_on_sc=False, needs_layout_passes=False
    ),
    out_type=jax.ShapeDtypeStruct((NE,), jnp.float32),
    scratch_types=[
        pltpu.VMEM((EPW,), jnp.int32),
        pltpu.VMEM((EPW,), jnp.int32),
        pltpu.VMEM((C, W), jnp.int32),
        pltpu.VMEM((C, W), jnp.int32),
        pltpu.VMEM((C, W), jnp.int32),
        pltpu.VMEM((C, W), jnp.int32),
        pltpu.VMEM((EPW,), jnp.float32),
        pltpu.SemaphoreType.DMA,
        pltpu.SemaphoreType.DMA,
        pltpu.SemaphoreType.DMA,
        pltpu.SemaphoreType.DMA,
    ],
)
def _score_kernel(table_hbm, ei_hbm, out_hbm, sidx, didx, sa, da, sb, db,
                  sc, sem_sa, sem_da, sem_sb, sem_db):
    wid = lax.axis_index("s") * NC + lax.axis_index("c")
    ebase = wid * EPW
    lane = lax.iota(jnp.int32, L)

    pltpu.sync_copy(ei_hbm.at[pl.ds(ebase, EPW)], sidx)
    pltpu.sync_copy(ei_hbm.at[pl.ds(NE + ebase, EPW)], didx)

    def issue(ci, srows, drows, sem_s, sem_d):
        base = ci * C
        pltpu.async_copy(table_hbm.at[sidx.at[pl.ds(base, H)]],
                         srows.at[pl.ds(0, H)], sem_s)
        pltpu.async_copy(table_hbm.at[sidx.at[pl.ds(base + H, H)]],
                         srows.at[pl.ds(H, H)], sem_s)
        pltpu.async_copy(table_hbm.at[didx.at[pl.ds(base, H)]],
                         drows.at[pl.ds(0, H)], sem_d)
        pltpu.async_copy(table_hbm.at[didx.at[pl.ds(base + H, H)]],
                         drows.at[pl.ds(H, H)], sem_d)

    def wait(ci, srows, drows, sem_s, sem_d):
        base = ci * C
        pltpu.make_async_copy(table_hbm.at[sidx.at[pl.ds(base, H)]],
                              srows.at[pl.ds(0, H)], sem_s).wait()
        pltpu.make_async_copy(table_hbm.at[sidx.at[pl.ds(base + H, H)]],
                              srows.at[pl.ds(H, H)], sem_s).wait()
        pltpu.make_async_copy(table_hbm.at[didx.at[pl.ds(base, H)]],
                              drows.at[pl.ds(0, H)], sem_d).wait()
        pltpu.make_async_copy(table_hbm.at[didx.at[pl.ds(base + H, H)]],
                              drows.at[pl.ds(H, H)], sem_d).wait()

    def compute(ci, srows, drows):
        def grp(g, c):
            rows = g * L + lane

            def prod(w):
                col = w ^ lane
                s32 = plsc.load_gather(srows, [rows, col])
                t32 = plsc.load_gather(drows, [rows, col])
                sbf = plsc.bitcast(s32, jnp.bfloat16)
                tbf = plsc.bitcast(t32, jnp.bfloat16)
                p_a, p_b = plsc.unpack(sbf * tbf,
                                       format=plsc.PackFormat.INTERLEAVED)
                return p_a + p_b

            def wstep(w4, accs):
                a0, a1, a2, a3 = accs
                w = w4 * 4
                return (a0 + prod(w), a1 + prod(w + 1),
                        a2 + prod(w + 2), a3 + prod(w + 3))

            z = jnp.zeros((L,), jnp.float32)
            a0, a1, a2, a3 = lax.fori_loop(0, W // 4, wstep, (z, z, z, z),
                                           unroll=4)
            sc[pl.ds(ci * C + g * L, L)] = (a0 + a1) + (a2 + a3)
            return c

        lax.fori_loop(0, G, grp, 0)

    issue(0, sa, da, sem_sa, sem_da)

    def pair(k, carry):
        c0 = 2 * k
        issue(c0 + 1, sb, db, sem_sb, sem_db)
        wait(c0, sa, da, sem_sa, sem_da)
        compute(c0, sa, da)
        issue(c0 + 2, sa, da, sem_sa, sem_da)
        wait(c0 + 1, sb, db, sem_sb, sem_db)
        compute(c0 + 1, sb, db)
        return carry

    lax.fori_loop(0, (NCHUNK - 1) // 2, pair, 0)
    wait(NCHUNK - 1, sa, da, sem_sa, sem_da)
    compute(NCHUNK - 1, sa, da)

    pltpu.sync_copy(sc, out_hbm.at[pl.ds(ebase, EPW)])


def kernel(gnn_emb, edge_index):
    mu16 = gnn_emb[:, :D].astype(jnp.bfloat16)
    table = lax.bitcast_convert_type(mu16.reshape(NV, W, 2), jnp.int32)
    ei = edge_index.reshape(2 * NE)
    return _score_kernel(table, ei)
